# baseline (speedup 1.0000x reference)
import dataclasses
import functools

import jax
import jax.numpy as jnp
from jax import lax
from jax.experimental import pallas as pl
from jax.experimental.pallas import tpu as pltpu
from jax.experimental.pallas import tpu_sc as plsc

N = 10000
E = 320000
IN_DIM = 128
OUT_DIM = 128
NUM_REL = 20
NUM_BASES = 10

NUM_CORES = 2
NUM_TILES = 16
LANES = 16
NUM_WORKERS = NUM_CORES * NUM_TILES

CH = 80
EDGES_PER_WORKER = E // NUM_WORKERS
MAIN_CHUNKS = EDGES_PER_WORKER // CH
CCH = 128
CNT_TOT_ROWS = E // CCH
CNT_ROWS = CNT_TOT_ROWS // NUM_TILES
CNT_EXTRA = CNT_TOT_ROWS - NUM_TILES * CNT_ROWS
CNT_PAD = 200704
CNT_PER_TILE = CNT_PAD // NUM_TILES
ZCNT = 3136
ROW_BLOCKS = N // CH

ROW_BLK = 1000
NUM_ROW_BLKS = N // ROW_BLK


def _tc_transform_body(comp_ref, w_ref, x_ref, root_ref, bias_ref,
                       y_ref, rout_ref, wcat_ref):
    nb = pl.program_id(0)

    @pl.when(nb == 0)
    def _():
        w = w_ref[...]
        for r in range(NUM_REL):
            wr = comp_ref[r, 0] * w[0]
            for b in range(1, NUM_BASES):
                wr = wr + comp_ref[r, b] * w[b]
            wcat_ref[:, r * OUT_DIM:(r + 1) * OUT_DIM] = wr

    x_blk = x_ref[...]
    y_ref[...] = jnp.dot(x_blk, wcat_ref[...],
                         preferred_element_type=jnp.float32,
                         precision=lax.Precision.HIGHEST)
    rout_ref[...] = jnp.dot(x_blk, root_ref[...],
                            preferred_element_type=jnp.float32,
                            precision=lax.Precision.HIGHEST) + bias_ref[...]


def _tc_transform(x, comp, weight, root, bias2d):
    y2, rout = pl.pallas_call(
        _tc_transform_body,
        grid=(NUM_ROW_BLKS,),
        in_specs=[
            pl.BlockSpec((NUM_REL, NUM_BASES), lambda nb: (0, 0),
                         memory_space=pltpu.SMEM),
            pl.BlockSpec((NUM_BASES, IN_DIM, OUT_DIM), lambda nb: (0, 0, 0)),
            pl.BlockSpec((ROW_BLK, IN_DIM), lambda nb: (nb, 0)),
            pl.BlockSpec((IN_DIM, OUT_DIM), lambda nb: (0, 0)),
            pl.BlockSpec((1, OUT_DIM), lambda nb: (0, 0)),
        ],
        out_specs=[
            pl.BlockSpec((ROW_BLK, NUM_REL * OUT_DIM), lambda nb: (nb, 0)),
            pl.BlockSpec((ROW_BLK, OUT_DIM), lambda nb: (nb, 0)),
        ],
        out_shape=[
            jax.ShapeDtypeStruct((N, NUM_REL * OUT_DIM), jnp.float32),
            jax.ShapeDtypeStruct((N, OUT_DIM), jnp.float32),
        ],
        scratch_shapes=[pltpu.VMEM((IN_DIM, NUM_REL * OUT_DIM), jnp.float32)],
    )(comp, weight, x, root, bias2d)
    return y2.reshape(N * NUM_REL, OUT_DIM), rout


def _sc_agg_body(y_hbm, pk_hbm, k2r_hbm, out_hbm,
                 counts_sh, acc_sh,
                 pk_v, cbuf_v, cnt_v, alpha_v, rows_v, zcnt_v, ones_v):
    c = lax.axis_index("c")
    s = lax.axis_index("s")
    wid = c * NUM_TILES + s

    zf32 = jnp.zeros((LANES,), jnp.float32)

    @pl.loop(0, ZCNT, step=LANES)
    def _(i):
        zcnt_v[pl.ds(i, LANES)] = zf32

    @pl.loop(0, CH)
    def _(i):
        for jj in range(0, OUT_DIM, LANES):
            rows_v[0][i, pl.ds(jj, LANES)] = zf32

    @pl.loop(0, CCH, step=LANES)
    def _(i):
        ones_v[pl.ds(i, LANES)] = jnp.ones((LANES,), jnp.float32)

    for rep in range(CNT_PER_TILE // ZCNT):
        pltpu.sync_copy(
            zcnt_v,
            counts_sh.at[pl.ds(s * CNT_PER_TILE + rep * ZCNT, ZCNT)])

    @pl.loop(0, 8)
    def _(j):
        blk = s + NUM_TILES * j

        @pl.when(blk < ROW_BLOCKS)
        def _():
            pltpu.sync_copy(rows_v[0], acc_sh.at[pl.ds(blk * CH, CH), :])

    plsc.subcore_barrier()

    def _phase1(ci0, ci1, ci2, ci3, ci4, ci5, cs0, cs1, cs2, cs3, cs4, cs5):
        cis = [ci0, ci1, ci2, ci3, ci4, ci5]
        css = [cs0, cs1, cs2, cs3, cs4, cs5]
        r0 = s * CNT_ROWS

        def ci(row, b):
            return pltpu.make_async_copy(k2r_hbm.at[row], cbuf_v[b], cis[b])

        def cs(b):
            return pltpu.make_async_copy(
                counts_sh.at[cbuf_v[b]], ones_v, css[b])

        def cs_start(b):
            pltpu.async_copy(ones_v, counts_sh.at[cbuf_v[b]], css[b],
                             add=True)

        def cs_wait(b):
            pltpu.make_async_copy(ones_v, counts_sh.at[cbuf_v[b]],
                                  css[b]).wait()

        ci(r0, 0).start()
        ci(r0 + 1, 1).start()

        @pl.loop(0, CNT_ROWS // 6)
        def _(k):
            for p in range(6):
                m = 6 * k + p
                ci(r0 + m, p).wait()
                cs_start(p)
                if p < 4:
                    @pl.when(k > 0)
                    def _():
                        cs_wait((p + 2) % 6)
                else:
                    cs_wait((p + 2) % 6)

                if p >= 4:
                    @pl.when(k < CNT_ROWS // 6 - 1)
                    def _():
                        ci(r0 + m + 2, (p + 2) % 6).start()
                else:
                    ci(r0 + m + 2, (p + 2) % 6).start()

        for m in range(CNT_ROWS - 4, CNT_ROWS):
            cs_wait(m % 6)

        @pl.when(s < CNT_EXTRA)
        def _():
            pltpu.sync_copy(k2r_hbm.at[NUM_TILES * CNT_ROWS + s], cbuf_v[0])
            pltpu.sync_copy(ones_v, counts_sh.at[cbuf_v[0]], add=True)

    pl.run_scoped(_phase1, *[pltpu.SemaphoreType.DMA(())
                             for _ in range(12)])

    plsc.subcore_barrier()

    def _phase2(sem0, sem1, rs0, rs1):
        psem = [sem0, sem1]
        rsem = [rs0, rs1]
        g0 = wid * MAIN_CHUNKS

        def pkc(g, b):
            return pltpu.make_async_copy(pk_hbm.at[g], pk_v[b], psem[b])

        def rowc(b):
            return pltpu.make_async_copy(y_hbm.at[pk_v[b].at[0]], rows_v[b],
                                         rsem[b])

        def pre(b):
            rowc(b).start()
            pltpu.sync_copy(counts_sh.at[pk_v[b].at[1]], cnt_v)

            @pl.loop(0, CH, step=LANES)
            def _(j):
                cv = cnt_v[pl.ds(j, LANES)]
                alpha_v[b][pl.ds(j, LANES)] = 1.0 / jnp.maximum(cv, 1.0)

        def fin(b):
            rowc(b).wait()

            @plsc.parallel_loop(0, CH, unroll=4)
            def _(e):
                av = plsc.load_gather(alpha_v[b],
                                      [jnp.zeros((LANES,), jnp.int32) + e])
                for jj in range(0, OUT_DIM, LANES):
                    rows_v[b][e, pl.ds(jj, LANES)] = (
                        rows_v[b][e, pl.ds(jj, LANES)] * av)

            pltpu.sync_copy(rows_v[b], acc_sh.at[pk_v[b].at[2]], add=True)

        pkc(g0, 0).start()
        pkc(g0, 0).wait()
        pre(0)
        pkc(g0 + 1, 1).start()

        @pl.loop(0, MAIN_CHUNKS // 2)
        def _(k):
            for b, ob in ((0, 1), (1, 0)):
                m = g0 + 2 * k + b
                pkc(m + 1, ob).wait()
                pre(ob)
                fin(b)

                @pl.when(m + 2 < g0 + MAIN_CHUNKS)
                def _():
                    pkc(m + 2, b).start()

        fin(0)

    pl.run_scoped(_phase2, pltpu.SemaphoreType.DMA(()),
                  pltpu.SemaphoreType.DMA(()), pltpu.SemaphoreType.DMA(()),
                  pltpu.SemaphoreType.DMA(()))

    plsc.subcore_barrier()

    @pl.loop(0, 8)
    def _(j):
        blk = s + NUM_TILES * j

        @pl.when(blk < ROW_BLOCKS)
        def _():
            pltpu.sync_copy(acc_sh.at[pl.ds(blk * CH, CH), :],
                            out_hbm.at[c, pl.ds(blk * CH, CH), :])


def _sc_aggregate(y, pk, k2r):
    mesh = plsc.VectorSubcoreMesh(core_axis_name="c", subcore_axis_name="s")
    cp = pltpu.CompilerParams()
    if "needs_layout_passes" in pltpu.CompilerParams.__dataclass_fields__:
        cp = dataclasses.replace(cp, needs_layout_passes=False)
    kern = pl.kernel(
        _sc_agg_body,
        compiler_params=cp,
        out_type=jax.ShapeDtypeStruct((NUM_CORES, N, OUT_DIM), jnp.float32),
        mesh=mesh,
        scratch_types=[
            pltpu.VMEM_SHARED((CNT_PAD,), jnp.float32),
            pltpu.VMEM_SHARED((N, OUT_DIM), jnp.float32),
            [pltpu.VMEM((3, CH), jnp.int32) for _ in range(2)],
            [pltpu.VMEM((CCH,), jnp.int32) for _ in range(6)],
            pltpu.VMEM((CH,), jnp.float32),
            [pltpu.VMEM((CH,), jnp.float32) for _ in range(2)],
            [pltpu.VMEM((CH, OUT_DIM), jnp.float32) for _ in range(2)],
            pltpu.VMEM((ZCNT,), jnp.float32),
            pltpu.VMEM((CCH,), jnp.float32),
        ],
    )
    return kern(y, pk, k2r)


def _tc_combine_body(p_ref, rout_ref, out_ref):
    out_ref[...] = p_ref[0] + p_ref[1] + rout_ref[...]


def _tc_combine(partials, rout):
    return pl.pallas_call(
        _tc_combine_body,
        out_shape=jax.ShapeDtypeStruct((N, OUT_DIM), jnp.float32),
    )(partials, rout)


@jax.jit
def kernel(x, edge_index, edge_type, comp, weight, root, bias):
    src = edge_index[0]
    dst = edge_index[1]
    k1 = src * NUM_REL + edge_type
    k2 = dst * NUM_REL + edge_type
    pk = jnp.stack([k1.reshape(-1, CH), k2.reshape(-1, CH),
                    dst.reshape(-1, CH)], axis=1)
    k2r = k2.reshape(-1, CCH)
    y, rout = _tc_transform(x, comp, weight, root, bias.reshape(1, OUT_DIM))
    partials = _sc_aggregate(y, pk, k2r)
    return _tc_combine(partials, rout)


# speedup vs baseline: 1.3202x; 1.3202x over previous
import dataclasses
import functools

import jax
import jax.numpy as jnp
from jax import lax
from jax.experimental import pallas as pl
from jax.experimental.pallas import tpu as pltpu
from jax.experimental.pallas import tpu_sc as plsc

N = 10000
E = 320000
IN_DIM = 128
OUT_DIM = 128
NUM_REL = 20
NUM_BASES = 10

NUM_CORES = 2
NUM_TILES = 16
LANES = 16
NUM_WORKERS = NUM_CORES * NUM_TILES

CH = 80
EDGES_PER_WORKER = E // NUM_WORKERS
MAIN_CHUNKS = EDGES_PER_WORKER // CH
CCH = 128
CNT_TOT_ROWS = E // CCH
CNT_ROWS = CNT_TOT_ROWS // NUM_TILES
CNT_EXTRA = CNT_TOT_ROWS - NUM_TILES * CNT_ROWS
CNT_PAD = 200704
CNT_PER_TILE = CNT_PAD // NUM_TILES
ZCNT = 3136
ROW_BLOCKS = N // CH

ROW_BLK = 2000
NUM_ROW_BLKS = N // ROW_BLK


def _tc_transform_body(comp_ref, w_ref, x_ref, root_ref, bias_ref,
                       y_ref, rout_ref):
    r = pl.program_id(1)
    w = w_ref[...]
    wr = comp_ref[r, 0] * w[0]
    for b in range(1, NUM_BASES):
        wr = wr + comp_ref[r, b] * w[b]
    y_ref[...] = jnp.dot(x_ref[...], wr,
                         preferred_element_type=jnp.float32)

    @pl.when(r == 0)
    def _():
        rout_ref[...] = jnp.dot(x_ref[...], root_ref[...],
                                preferred_element_type=jnp.float32) + bias_ref[...]


def _tc_transform(x, comp, weight, root, bias2d):
    return pl.pallas_call(
        _tc_transform_body,
        grid=(NUM_ROW_BLKS, NUM_REL),
        in_specs=[
            pl.BlockSpec((NUM_REL, NUM_BASES), lambda nb, r: (0, 0),
                         memory_space=pltpu.SMEM),
            pl.BlockSpec((NUM_BASES, IN_DIM, OUT_DIM), lambda nb, r: (0, 0, 0)),
            pl.BlockSpec((ROW_BLK, IN_DIM), lambda nb, r: (nb, 0)),
            pl.BlockSpec((IN_DIM, OUT_DIM), lambda nb, r: (0, 0)),
            pl.BlockSpec((1, OUT_DIM), lambda nb, r: (0, 0)),
        ],
        out_specs=[
            pl.BlockSpec((ROW_BLK, OUT_DIM),
                         lambda nb, r: (r * NUM_ROW_BLKS + nb, 0)),
            pl.BlockSpec((ROW_BLK, OUT_DIM), lambda nb, r: (nb, 0)),
        ],
        out_shape=[
            jax.ShapeDtypeStruct((NUM_REL * N, OUT_DIM), jnp.float32),
            jax.ShapeDtypeStruct((N, OUT_DIM), jnp.float32),
        ],
    )(comp, weight, x, root, bias2d)


def _sc_agg_body(y_hbm, pk_hbm, k2r_hbm, out_hbm,
                 counts_sh, acc_sh,
                 pk_v, cbuf_v, cnt_v, alpha_v, rows_v, zcnt_v, ones_v):
    c = lax.axis_index("c")
    s = lax.axis_index("s")
    wid = c * NUM_TILES + s

    zf32 = jnp.zeros((LANES,), jnp.float32)

    @pl.loop(0, ZCNT, step=LANES)
    def _(i):
        zcnt_v[pl.ds(i, LANES)] = zf32

    @pl.loop(0, CH)
    def _(i):
        for jj in range(0, OUT_DIM, LANES):
            rows_v[0][i, pl.ds(jj, LANES)] = zf32

    @pl.loop(0, CCH, step=LANES)
    def _(i):
        ones_v[pl.ds(i, LANES)] = jnp.ones((LANES,), jnp.float32)

    for rep in range(CNT_PER_TILE // ZCNT):
        pltpu.sync_copy(
            zcnt_v,
            counts_sh.at[pl.ds(s * CNT_PER_TILE + rep * ZCNT, ZCNT)])

    @pl.loop(0, 8)
    def _(j):
        blk = s + NUM_TILES * j

        @pl.when(blk < ROW_BLOCKS)
        def _():
            pltpu.sync_copy(rows_v[0], acc_sh.at[pl.ds(blk * CH, CH), :])

    plsc.subcore_barrier()

    def _phase1(ci0, ci1, ci2, ci3, ci4, ci5, cs0, cs1, cs2, cs3, cs4, cs5):
        cis = [ci0, ci1, ci2, ci3, ci4, ci5]
        css = [cs0, cs1, cs2, cs3, cs4, cs5]
        r0 = s * CNT_ROWS

        def ci(row, b):
            return pltpu.make_async_copy(k2r_hbm.at[row], cbuf_v[b], cis[b])

        def cs(b):
            return pltpu.make_async_copy(
                counts_sh.at[cbuf_v[b]], ones_v, css[b])

        def cs_start(b):
            pltpu.async_copy(ones_v, counts_sh.at[cbuf_v[b]], css[b],
                             add=True)

        def cs_wait(b):
            pltpu.make_async_copy(ones_v, counts_sh.at[cbuf_v[b]],
                                  css[b]).wait()

        ci(r0, 0).start()
        ci(r0 + 1, 1).start()

        @pl.loop(0, CNT_ROWS // 6)
        def _(k):
            for p in range(6):
                m = 6 * k + p
                ci(r0 + m, p).wait()
                cs_start(p)
                if p < 4:
                    @pl.when(k > 0)
                    def _():
                        cs_wait((p + 2) % 6)
                else:
                    cs_wait((p + 2) % 6)

                if p >= 4:
                    @pl.when(k < CNT_ROWS // 6 - 1)
                    def _():
                        ci(r0 + m + 2, (p + 2) % 6).start()
                else:
                    ci(r0 + m + 2, (p + 2) % 6).start()

        for m in range(CNT_ROWS - 4, CNT_ROWS):
            cs_wait(m % 6)

        @pl.when(s < CNT_EXTRA)
        def _():
            pltpu.sync_copy(k2r_hbm.at[NUM_TILES * CNT_ROWS + s], cbuf_v[0])
            pltpu.sync_copy(ones_v, counts_sh.at[cbuf_v[0]], add=True)

    pl.run_scoped(_phase1, *[pltpu.SemaphoreType.DMA(())
                             for _ in range(12)])

    plsc.subcore_barrier()

    def _phase2(sem0, sem1, rs0, rs1):
        psem = [sem0, sem1]
        rsem = [rs0, rs1]
        g0 = wid * MAIN_CHUNKS

        def pkc(g, b):
            return pltpu.make_async_copy(pk_hbm.at[g], pk_v[b], psem[b])

        def rowc(b):
            return pltpu.make_async_copy(y_hbm.at[pk_v[b].at[0]], rows_v[b],
                                         rsem[b])

        def pre(b):
            rowc(b).start()
            pltpu.sync_copy(counts_sh.at[pk_v[b].at[1]], cnt_v)

            @pl.loop(0, CH, step=LANES)
            def _(j):
                cv = cnt_v[pl.ds(j, LANES)]
                alpha_v[b][pl.ds(j, LANES)] = 1.0 / jnp.maximum(cv, 1.0)

        def fin(b):
            rowc(b).wait()

            @plsc.parallel_loop(0, CH, unroll=4)
            def _(e):
                av = plsc.load_gather(alpha_v[b],
                                      [jnp.zeros((LANES,), jnp.int32) + e])
                for jj in range(0, OUT_DIM, LANES):
                    rows_v[b][e, pl.ds(jj, LANES)] = (
                        rows_v[b][e, pl.ds(jj, LANES)] * av)

            pltpu.sync_copy(rows_v[b], acc_sh.at[pk_v[b].at[2]], add=True)

        pkc(g0, 0).start()
        pkc(g0, 0).wait()
        pre(0)
        pkc(g0 + 1, 1).start()

        @pl.loop(0, MAIN_CHUNKS // 2)
        def _(k):
            for b, ob in ((0, 1), (1, 0)):
                m = g0 + 2 * k + b
                pkc(m + 1, ob).wait()
                pre(ob)
                fin(b)

                @pl.when(m + 2 < g0 + MAIN_CHUNKS)
                def _():
                    pkc(m + 2, b).start()

        fin(0)

    pl.run_scoped(_phase2, pltpu.SemaphoreType.DMA(()),
                  pltpu.SemaphoreType.DMA(()), pltpu.SemaphoreType.DMA(()),
                  pltpu.SemaphoreType.DMA(()))

    plsc.subcore_barrier()

    @pl.loop(0, 8)
    def _(j):
        blk = s + NUM_TILES * j

        @pl.when(blk < ROW_BLOCKS)
        def _():
            pltpu.sync_copy(acc_sh.at[pl.ds(blk * CH, CH), :],
                            out_hbm.at[c, pl.ds(blk * CH, CH), :])


def _sc_aggregate(y, pk, k2r):
    mesh = plsc.VectorSubcoreMesh(core_axis_name="c", subcore_axis_name="s")
    cp = pltpu.CompilerParams()
    if "needs_layout_passes" in pltpu.CompilerParams.__dataclass_fields__:
        cp = dataclasses.replace(cp, needs_layout_passes=False)
    kern = pl.kernel(
        _sc_agg_body,
        compiler_params=cp,
        out_type=jax.ShapeDtypeStruct((NUM_CORES, N, OUT_DIM), jnp.float32),
        mesh=mesh,
        scratch_types=[
            pltpu.VMEM_SHARED((CNT_PAD,), jnp.float32),
            pltpu.VMEM_SHARED((N, OUT_DIM), jnp.float32),
            [pltpu.VMEM((3, CH), jnp.int32) for _ in range(2)],
            [pltpu.VMEM((CCH,), jnp.int32) for _ in range(6)],
            pltpu.VMEM((CH,), jnp.float32),
            [pltpu.VMEM((CH,), jnp.float32) for _ in range(2)],
            [pltpu.VMEM((CH, OUT_DIM), jnp.float32) for _ in range(2)],
            pltpu.VMEM((ZCNT,), jnp.float32),
            pltpu.VMEM((CCH,), jnp.float32),
        ],
    )
    return kern(y, pk, k2r)


def _tc_combine_body(p_ref, rout_ref, out_ref):
    out_ref[...] = p_ref[0] + p_ref[1] + rout_ref[...]


def _tc_combine(partials, rout):
    return pl.pallas_call(
        _tc_combine_body,
        out_shape=jax.ShapeDtypeStruct((N, OUT_DIM), jnp.float32),
    )(partials, rout)


@jax.jit
def kernel(x, edge_index, edge_type, comp, weight, root, bias):
    src = edge_index[0]
    dst = edge_index[1]
    k1 = edge_type * N + src
    k2 = edge_type * N + dst
    pk = jnp.stack([k1.reshape(-1, CH), k2.reshape(-1, CH),
                    dst.reshape(-1, CH)], axis=1)
    k2r = k2.reshape(-1, CCH)
    y, rout = _tc_transform(x, comp, weight, root, bias.reshape(1, OUT_DIM))
    partials = _sc_aggregate(y, pk, k2r)
    return _tc_combine(partials, rout)


# speedup vs baseline: 1.5254x; 1.1554x over previous
import dataclasses
import functools

import jax
import jax.numpy as jnp
from jax import lax
from jax.experimental import pallas as pl
from jax.experimental.pallas import tpu as pltpu
from jax.experimental.pallas import tpu_sc as plsc

N = 10000
E = 320000
IN_DIM = 128
OUT_DIM = 128
NUM_REL = 20
NUM_BASES = 10

NUM_CORES = 2
NUM_TILES = 16
LANES = 16
NUM_WORKERS = NUM_CORES * NUM_TILES

CH = 80
EDGES_PER_WORKER = E // NUM_WORKERS
MAIN_CHUNKS = EDGES_PER_WORKER // CH
CCH = 128
CNT_TOT_ROWS = E // CCH
CNT_ROWS = CNT_TOT_ROWS // NUM_TILES
CNT_EXTRA = CNT_TOT_ROWS - NUM_TILES * CNT_ROWS
CNT_PAD = 200704
CNT_PER_TILE = CNT_PAD // NUM_TILES
ZCNT = 3136
ROW_BLOCKS = N // CH

ROW_BLK = 2000
NUM_ROW_BLKS = N // ROW_BLK


def _tc_transform_body(comp_ref, w_ref, x_ref, root_ref, bias_ref,
                       y_ref, rout_ref):
    r = pl.program_id(1)
    w = w_ref[...]
    wr = comp_ref[r, 0] * w[0]
    for b in range(1, NUM_BASES):
        wr = wr + comp_ref[r, b] * w[b]
    y_ref[...] = jnp.dot(x_ref[...], wr,
                         preferred_element_type=jnp.float32)

    @pl.when(r == 0)
    def _():
        rout_ref[...] = jnp.dot(x_ref[...], root_ref[...],
                                preferred_element_type=jnp.float32) + bias_ref[...]


def _tc_transform(x, comp, weight, root, bias2d):
    return pl.pallas_call(
        _tc_transform_body,
        grid=(NUM_ROW_BLKS, NUM_REL),
        in_specs=[
            pl.BlockSpec((NUM_REL, NUM_BASES), lambda nb, r: (0, 0),
                         memory_space=pltpu.SMEM),
            pl.BlockSpec((NUM_BASES, IN_DIM, OUT_DIM), lambda nb, r: (0, 0, 0)),
            pl.BlockSpec((ROW_BLK, IN_DIM), lambda nb, r: (nb, 0)),
            pl.BlockSpec((IN_DIM, OUT_DIM), lambda nb, r: (0, 0)),
            pl.BlockSpec((1, OUT_DIM), lambda nb, r: (0, 0)),
        ],
        out_specs=[
            pl.BlockSpec((ROW_BLK, OUT_DIM),
                         lambda nb, r: (r * NUM_ROW_BLKS + nb, 0)),
            pl.BlockSpec((ROW_BLK, OUT_DIM), lambda nb, r: (nb, 0)),
        ],
        out_shape=[
            jax.ShapeDtypeStruct((NUM_REL * N, OUT_DIM), jnp.float32),
            jax.ShapeDtypeStruct((N, OUT_DIM), jnp.float32),
        ],
    )(comp, weight, x, root, bias2d)


def _sc_agg_body(y_hbm, pk_hbm, k2r_hbm, out_hbm,
                 counts_sh, acc_sh,
                 pk_v, cbuf_v, cnt_v, alpha_v, rows_v, zcnt_v, ones_v):
    c = lax.axis_index("c")
    s = lax.axis_index("s")
    wid = c * NUM_TILES + s

    zf32 = jnp.zeros((LANES,), jnp.float32)

    @pl.loop(0, ZCNT, step=LANES)
    def _(i):
        zcnt_v[pl.ds(i, LANES)] = zf32

    @pl.loop(0, CH)
    def _(i):
        for jj in range(0, OUT_DIM, LANES):
            rows_v[0][i, pl.ds(jj, LANES)] = zf32

    @pl.loop(0, CCH, step=LANES)
    def _(i):
        ones_v[pl.ds(i, LANES)] = jnp.ones((LANES,), jnp.float32)

    for rep in range(CNT_PER_TILE // ZCNT):
        pltpu.sync_copy(
            zcnt_v,
            counts_sh.at[pl.ds(s * CNT_PER_TILE + rep * ZCNT, ZCNT)])

    @pl.loop(0, 8)
    def _(j):
        blk = s + NUM_TILES * j

        @pl.when(blk < ROW_BLOCKS)
        def _():
            pltpu.sync_copy(rows_v[0], acc_sh.at[pl.ds(blk * CH, CH), :])

    plsc.subcore_barrier()

    def _phase1(ci0, ci1, ci2, ci3, ci4, ci5, cs0, cs1, cs2, cs3, cs4, cs5):
        cis = [ci0, ci1, ci2, ci3, ci4, ci5]
        css = [cs0, cs1, cs2, cs3, cs4, cs5]
        r0 = s * CNT_ROWS

        def ci(row, b):
            return pltpu.make_async_copy(k2r_hbm.at[row], cbuf_v[b], cis[b])

        def cs(b):
            return pltpu.make_async_copy(
                counts_sh.at[cbuf_v[b]], ones_v, css[b])

        def cs_start(b):
            pltpu.async_copy(ones_v, counts_sh.at[cbuf_v[b]], css[b],
                             add=True)

        def cs_wait(b):
            pltpu.make_async_copy(ones_v, counts_sh.at[cbuf_v[b]],
                                  css[b]).wait()

        ci(r0, 0).start()
        ci(r0 + 1, 1).start()

        @pl.loop(0, CNT_ROWS // 6)
        def _(k):
            for p in range(6):
                m = 6 * k + p
                ci(r0 + m, p).wait()
                cs_start(p)
                if p < 4:
                    @pl.when(k > 0)
                    def _():
                        cs_wait((p + 2) % 6)
                else:
                    cs_wait((p + 2) % 6)

                if p >= 4:
                    @pl.when(k < CNT_ROWS // 6 - 1)
                    def _():
                        ci(r0 + m + 2, (p + 2) % 6).start()
                else:
                    ci(r0 + m + 2, (p + 2) % 6).start()

        for m in range(CNT_ROWS - 4, CNT_ROWS):
            cs_wait(m % 6)

        @pl.when(s < CNT_EXTRA)
        def _():
            pltpu.sync_copy(k2r_hbm.at[NUM_TILES * CNT_ROWS + s], cbuf_v[0])
            pltpu.sync_copy(ones_v, counts_sh.at[cbuf_v[0]], add=True)

    pl.run_scoped(_phase1, *[pltpu.SemaphoreType.DMA(())
                             for _ in range(12)])

    plsc.subcore_barrier()

    def _phase2(p0, p1, p2, r0, r1, r2, s0, s1, s2):
        psem = [p0, p1, p2]
        rsem = [r0, r1, r2]
        ssem = [s0, s1, s2]
        g0 = wid * MAIN_CHUNKS

        def pkc(g, b):
            return pltpu.make_async_copy(pk_hbm.at[g], pk_v[b], psem[b])

        def rowc(b):
            return pltpu.make_async_copy(y_hbm.at[pk_v[b].at[0]], rows_v[b],
                                         rsem[b])

        def scat_start(b):
            pltpu.async_copy(rows_v[b], acc_sh.at[pk_v[b].at[2]], ssem[b],
                             add=True)

        def scat_wait(b):
            pltpu.make_async_copy(rows_v[b], acc_sh.at[pk_v[b].at[2]],
                                  ssem[b]).wait()

        def pre(b):
            rowc(b).start()
            pltpu.sync_copy(counts_sh.at[pk_v[b].at[1]], cnt_v)

            @pl.loop(0, CH, step=LANES)
            def _(j):
                cv = cnt_v[pl.ds(j, LANES)]
                alpha_v[b][pl.ds(j, LANES)] = 1.0 / jnp.maximum(cv, 1.0)

        def fin(b):
            rowc(b).wait()

            @plsc.parallel_loop(0, CH, unroll=4)
            def _(e):
                av = plsc.load_gather(alpha_v[b],
                                      [jnp.zeros((LANES,), jnp.int32) + e])
                for jj in range(0, OUT_DIM, LANES):
                    rows_v[b][e, pl.ds(jj, LANES)] = (
                        rows_v[b][e, pl.ds(jj, LANES)] * av)

            scat_start(b)

        pkc(g0, 0).start()
        pkc(g0, 0).wait()
        pre(0)
        pkc(g0 + 1, 1).start()

        @pl.loop(0, 41)
        def _(k):
            for p in range(3):
                m = 3 * k + p
                pn = (p + 1) % 3
                pnn = (p + 2) % 3
                pkc(g0 + m + 1, pn).wait()
                pre(pn)
                fin(p)
                if p == 0:
                    @pl.when(k > 0)
                    def _():
                        scat_wait(pnn)
                else:
                    scat_wait(pnn)
                pkc(g0 + m + 2, pnn).start()

        pkc(g0 + MAIN_CHUNKS - 1, 1).wait()
        pre(1)
        fin(0)
        scat_wait(2)
        fin(1)
        scat_wait(0)
        scat_wait(1)

    pl.run_scoped(_phase2, *[pltpu.SemaphoreType.DMA(())
                             for _ in range(9)])

    plsc.subcore_barrier()

    @pl.loop(0, 8)
    def _(j):
        blk = s + NUM_TILES * j

        @pl.when(blk < ROW_BLOCKS)
        def _():
            pltpu.sync_copy(acc_sh.at[pl.ds(blk * CH, CH), :],
                            out_hbm.at[c, pl.ds(blk * CH, CH), :])


def _sc_aggregate(y, pk, k2r):
    mesh = plsc.VectorSubcoreMesh(core_axis_name="c", subcore_axis_name="s")
    cp = pltpu.CompilerParams()
    if "needs_layout_passes" in pltpu.CompilerParams.__dataclass_fields__:
        cp = dataclasses.replace(cp, needs_layout_passes=False)
    kern = pl.kernel(
        _sc_agg_body,
        compiler_params=cp,
        out_type=jax.ShapeDtypeStruct((NUM_CORES, N, OUT_DIM), jnp.float32),
        mesh=mesh,
        scratch_types=[
            pltpu.VMEM_SHARED((CNT_PAD,), jnp.float32),
            pltpu.VMEM_SHARED((N, OUT_DIM), jnp.float32),
            [pltpu.VMEM((3, CH), jnp.int32) for _ in range(3)],
            [pltpu.VMEM((CCH,), jnp.int32) for _ in range(6)],
            pltpu.VMEM((CH,), jnp.float32),
            [pltpu.VMEM((CH,), jnp.float32) for _ in range(3)],
            [pltpu.VMEM((CH, OUT_DIM), jnp.float32) for _ in range(3)],
            pltpu.VMEM((ZCNT,), jnp.float32),
            pltpu.VMEM((CCH,), jnp.float32),
        ],
    )
    return kern(y, pk, k2r)


def _tc_combine_body(p_ref, rout_ref, out_ref):
    out_ref[...] = p_ref[0] + p_ref[1] + rout_ref[...]


def _tc_combine(partials, rout):
    return pl.pallas_call(
        _tc_combine_body,
        out_shape=jax.ShapeDtypeStruct((N, OUT_DIM), jnp.float32),
    )(partials, rout)


@jax.jit
def kernel(x, edge_index, edge_type, comp, weight, root, bias):
    src = edge_index[0]
    dst = edge_index[1]
    k1 = edge_type * N + src
    k2 = edge_type * N + dst
    pk = jnp.stack([k1.reshape(-1, CH), k2.reshape(-1, CH),
                    dst.reshape(-1, CH)], axis=1)
    k2r = k2.reshape(-1, CCH)
    y, rout = _tc_transform(x, comp, weight, root, bias.reshape(1, OUT_DIM))
    partials = _sc_aggregate(y, pk, k2r)
    return _tc_combine(partials, rout)
